# baseline (speedup 1.0000x reference)
import functools

import jax
import jax.numpy as jnp
from jax import lax
from jax.experimental import pallas as pl
from jax.experimental.pallas import tpu as pltpu
from jax.experimental.pallas import tpu_sc as plsc

_CHUNK = 128


def _sc_gather(idx, emb, *, nw, chunks):
    nrows = idx.shape[0]
    embed_dim = emb.shape[1]
    mesh = plsc.VectorSubcoreMesh(core_axis_name="c", subcore_axis_name="s")
    num_cores = mesh.num_cores

    @functools.partial(
        pl.kernel,
        out_type=jax.ShapeDtypeStruct((nrows, embed_dim), jnp.float32),
        mesh=mesh,
        scratch_types=[
            pltpu.VMEM((chunks * _CHUNK,), jnp.int32),
            pltpu.VMEM((2, _CHUNK, embed_dim), jnp.float32),
            pltpu.SemaphoreType.DMA,
            pltpu.SemaphoreType.DMA,
        ],
        compiler_params=pltpu.CompilerParams(use_tc_tiling_on_sc=True),
    )
    def gather_kernel(emb_hbm, idx_hbm, out_hbm, idx_v, rows_v, sem0, sem1):
        wid = lax.axis_index("s") * num_cores + lax.axis_index("c")
        base = wid * (chunks * _CHUNK)
        pltpu.sync_copy(idx_hbm.at[pl.ds(base, chunks * _CHUNK)], idx_v)
        sems = (sem0, sem1)

        def start(g):
            return pltpu.async_copy(
                emb_hbm.at[idx_v.at[pl.ds(g * _CHUNK, _CHUNK)]],
                rows_v.at[g % 2],
                sems[g % 2],
            )

        pending = start(0)
        for g in range(chunks):
            nxt = start(g + 1) if g + 1 < chunks else None
            pending.wait()
            pltpu.sync_copy(
                rows_v.at[g % 2], out_hbm.at[pl.ds(base + g * _CHUNK, _CHUNK)]
            )
            pending = nxt

    return gather_kernel(emb, idx)


def _fuse_weights(W1, W2, b1, b2):
    def body(w1_ref, w2_ref, b1_ref, b2_ref, wf_ref, bf_ref):
        w2 = w2_ref[...]
        wf_ref[...] = jnp.dot(
            w1_ref[...], w2,
            preferred_element_type=jnp.float32,
            precision=lax.Precision.HIGHEST,
        )
        bf_ref[...] = (
            jnp.dot(
                b1_ref[...], w2,
                preferred_element_type=jnp.float32,
                precision=lax.Precision.HIGHEST,
            )
            + b2_ref[...]
        )

    embed_dim, hidden = W1.shape
    num_out = W2.shape[1]
    return pl.pallas_call(
        body,
        out_shape=(
            jax.ShapeDtypeStruct((embed_dim, num_out), jnp.float32),
            jax.ShapeDtypeStruct((1, num_out), jnp.float32),
        ),
    )(W1, W2, b1.reshape(1, hidden), b2.reshape(1, num_out))


def _mlp(gathered, wf, bf, *, block_rows):
    nrows, embed_dim = gathered.shape
    num_out = wf.shape[1]

    def body(x_ref, wf_ref, bf_ref, o_ref):
        o_ref[...] = (
            jnp.dot(x_ref[...], wf_ref[...], preferred_element_type=jnp.float32)
            + bf_ref[...]
        )

    return pl.pallas_call(
        body,
        grid=(nrows // block_rows,),
        in_specs=[
            pl.BlockSpec((block_rows, embed_dim), lambda i: (i, 0)),
            pl.BlockSpec((embed_dim, num_out), lambda i: (0, 0)),
            pl.BlockSpec((1, num_out), lambda i: (0, 0)),
        ],
        out_specs=pl.BlockSpec((block_rows, num_out), lambda i: (i, 0)),
        out_shape=jax.ShapeDtypeStruct((nrows, num_out), jnp.float32),
    )(gathered, wf, bf)


def kernel(input_notes, emb, W1, b1, W2, b2):
    batch, hist = input_notes.shape
    nrows = batch * hist
    info = plsc.get_sparse_core_info()
    nw = info.num_cores * info.num_subcores
    chunks = nrows // (nw * _CHUNK)
    idx = input_notes.reshape(nrows).astype(jnp.int32)

    gathered = _sc_gather(idx, emb, nw=nw, chunks=chunks)
    wf, bf = _fuse_weights(W1, W2, b1, b2)
    out = _mlp(gathered, wf, bf, block_rows=1024)
    return out.reshape(batch, hist, W2.shape[1])


# speedup vs baseline: 1.4774x; 1.4774x over previous
import functools

import jax
import jax.numpy as jnp
from jax import lax
from jax.experimental import pallas as pl
from jax.experimental.pallas import tpu as pltpu
from jax.experimental.pallas import tpu_sc as plsc

_CHUNK = 128
_LPAD = 24


def _sc_gather_padded(idx, sidx, emb, *, nw, chunks, hist):
    nrows = idx.shape[0]
    out_rows = (nrows // hist) * _LPAD
    embed_dim = emb.shape[1]
    pad_chunks = sidx.shape[1]
    mesh = plsc.VectorSubcoreMesh(core_axis_name="c", subcore_axis_name="s")
    num_cores = mesh.num_cores
    nbuf = 4

    @functools.partial(
        pl.kernel,
        out_type=jax.ShapeDtypeStruct((out_rows, embed_dim), jnp.float32),
        mesh=mesh,
        scratch_types=[
            pltpu.VMEM((chunks * _CHUNK,), jnp.int32),
            pltpu.VMEM((pad_chunks, _CHUNK), jnp.int32),
            pltpu.VMEM((nbuf, _CHUNK, embed_dim), jnp.float32),
            [pltpu.SemaphoreType.DMA] * nbuf,
            [pltpu.SemaphoreType.DMA] * nbuf,
        ],
    )
    def gather_kernel(emb_hbm, idx_hbm, sidx_hbm, out_hbm, idx_v, sidx_v,
                      rows_v, gsems, ssems):
        wid = lax.axis_index("s") * num_cores + lax.axis_index("c")
        base = wid * (chunks * _CHUNK)
        pltpu.sync_copy(idx_hbm.at[pl.ds(base, chunks * _CHUNK)], idx_v)
        pltpu.sync_copy(sidx_hbm.at[wid], sidx_v)

        def gather(g):
            return pltpu.async_copy(
                emb_hbm.at[idx_v.at[pl.ds(g * _CHUNK, _CHUNK)]],
                rows_v.at[g % nbuf],
                gsems[g % nbuf],
            )

        def scatter(g):
            return pltpu.async_copy(
                rows_v.at[g % nbuf],
                out_hbm.at[sidx_v.at[g]],
                ssems[g % nbuf],
            )

        gath = {g: gather(g) for g in range(min(2, chunks))}
        scat = {}
        for g in range(chunks):
            gath.pop(g).wait()
            scat[g] = scatter(g)
            nx = g + 2
            if nx < chunks:
                if nx - nbuf in scat:
                    scat.pop(nx - nbuf).wait()
                gath[nx] = gather(nx)
        for d in scat.values():
            d.wait()

    return gather_kernel(emb, idx, sidx)


def _fuse_weights(W1, W2, b1, b2):
    def body(w1_ref, w2_ref, b1_ref, b2_ref, wf_ref, bf_ref):
        w2 = w2_ref[...]
        wf_ref[...] = jnp.dot(
            w1_ref[...], w2,
            preferred_element_type=jnp.float32,
            precision=lax.Precision.HIGHEST,
        )
        bf_ref[...] = (
            jnp.dot(
                b1_ref[...], w2,
                preferred_element_type=jnp.float32,
                precision=lax.Precision.HIGHEST,
            )
            + b2_ref[...]
        )

    embed_dim, hidden = W1.shape
    num_out = W2.shape[1]
    return pl.pallas_call(
        body,
        out_shape=(
            jax.ShapeDtypeStruct((embed_dim, num_out), jnp.float32),
            jax.ShapeDtypeStruct((1, 1, num_out), jnp.float32),
        ),
    )(W1, W2, b1.reshape(1, hidden), b2.reshape(1, 1, num_out))


def _mlp(gathered3, wf, bf, *, block_b, hist):
    batch = gathered3.shape[0]
    embed_dim = gathered3.shape[2]
    num_out = wf.shape[1]

    def body(x_ref, wf_ref, bf_ref, o_ref):
        x = x_ref[...].reshape(block_b * _LPAD, embed_dim)
        y = jnp.dot(x, wf_ref[...], preferred_element_type=jnp.float32)
        y3 = y.reshape(block_b, _LPAD, num_out)
        o_ref[...] = y3[:, :hist, :] + bf_ref[...]

    return pl.pallas_call(
        body,
        grid=(batch // block_b,),
        in_specs=[
            pl.BlockSpec((block_b, _LPAD, embed_dim), lambda i: (i, 0, 0)),
            pl.BlockSpec((embed_dim, num_out), lambda i: (0, 0)),
            pl.BlockSpec((1, 1, num_out), lambda i: (0, 0, 0)),
        ],
        out_specs=pl.BlockSpec((block_b, hist, num_out), lambda i: (i, 0, 0)),
        out_shape=jax.ShapeDtypeStruct((batch, hist, num_out), jnp.float32),
    )(gathered3, wf, bf)


def kernel(input_notes, emb, W1, b1, W2, b2):
    batch, hist = input_notes.shape
    nrows = batch * hist
    info = plsc.get_sparse_core_info()
    nw = info.num_cores * info.num_subcores
    chunks = nrows // (nw * _CHUNK)

    idx = input_notes.reshape(nrows).astype(jnp.int32)
    rows = jnp.arange(nrows, dtype=jnp.int32)
    sidx = ((rows // hist) * _LPAD + rows % hist).reshape(nw, chunks, _CHUNK)
    sidx = jnp.pad(sidx, ((0, 0), (0, -chunks % 8), (0, 0)))

    gpad = _sc_gather_padded(idx, sidx, emb, nw=nw, chunks=chunks, hist=hist)
    wf, bf = _fuse_weights(W1, W2, b1, b2)
    out = _mlp(
        gpad.reshape(batch, _LPAD, emb.shape[1]), wf, bf,
        block_b=64, hist=hist,
    )
    return out


# speedup vs baseline: 4.4815x; 3.0334x over previous
import functools

import jax
import jax.numpy as jnp
from jax import lax
from jax.experimental import pallas as pl
from jax.experimental.pallas import tpu as pltpu
from jax.experimental.pallas import tpu_sc as plsc

_CHUNK = 128


def _sc_gather(idx, sidx, emb, *, chunks):
    nrows = idx.shape[0]
    embed_dim = emb.shape[1]
    pad_chunks = sidx.shape[1]
    mesh = plsc.VectorSubcoreMesh(core_axis_name="c", subcore_axis_name="s")
    num_cores = mesh.num_cores
    nbuf = 4

    @functools.partial(
        pl.kernel,
        out_type=jax.ShapeDtypeStruct((nrows, embed_dim), jnp.float32),
        mesh=mesh,
        scratch_types=[
            pltpu.VMEM((chunks * _CHUNK,), jnp.int32),
            pltpu.VMEM((pad_chunks, _CHUNK), jnp.int32),
            pltpu.VMEM((nbuf, _CHUNK, embed_dim), jnp.float32),
            [pltpu.SemaphoreType.DMA] * nbuf,
            [pltpu.SemaphoreType.DMA] * nbuf,
        ],
    )
    def gather_kernel(emb_hbm, idx_hbm, sidx_hbm, out_hbm, idx_v, sidx_v,
                      rows_v, gsems, ssems):
        wid = lax.axis_index("s") * num_cores + lax.axis_index("c")
        base = wid * (chunks * _CHUNK)
        pltpu.sync_copy(idx_hbm.at[pl.ds(base, chunks * _CHUNK)], idx_v)
        pltpu.sync_copy(sidx_hbm.at[wid], sidx_v)

        def gather(g):
            return pltpu.async_copy(
                emb_hbm.at[idx_v.at[pl.ds(g * _CHUNK, _CHUNK)]],
                rows_v.at[g % nbuf],
                gsems[g % nbuf],
            )

        def scatter(g):
            return pltpu.async_copy(
                rows_v.at[g % nbuf],
                out_hbm.at[sidx_v.at[g]],
                ssems[g % nbuf],
            )

        gath = {g: gather(g) for g in range(min(2, chunks))}
        scat = {}
        for g in range(chunks):
            gath.pop(g).wait()
            scat[g] = scatter(g)
            nx = g + 2
            if nx < chunks:
                if nx - nbuf in scat:
                    scat.pop(nx - nbuf).wait()
                gath[nx] = gather(nx)
        for d in scat.values():
            d.wait()

    return gather_kernel(emb, idx, sidx)


def _fuse_weights(W1, W2, b1, b2):
    def body(w1_ref, w2_ref, b1_ref, b2_ref, wft_ref, bft_ref):
        w2 = w2_ref[...]
        wft_ref[...] = lax.dot_general(
            w2, w1_ref[...],
            dimension_numbers=(((0,), (1,)), ((), ())),
            preferred_element_type=jnp.float32,
            precision=lax.Precision.HIGHEST,
        )
        bft_ref[...] = (
            lax.dot_general(
                w2, b1_ref[...],
                dimension_numbers=(((0,), (0,)), ((), ())),
                preferred_element_type=jnp.float32,
                precision=lax.Precision.HIGHEST,
            )
            + b2_ref[...]
        )

    embed_dim, hidden = W1.shape
    num_out = W2.shape[1]
    return pl.pallas_call(
        body,
        out_shape=(
            jax.ShapeDtypeStruct((num_out, embed_dim), jnp.float32),
            jax.ShapeDtypeStruct((num_out, 1), jnp.float32),
        ),
    )(W1, W2, b1.reshape(hidden, 1), b2.reshape(num_out, 1))


def _mlp_t(gathered, wft, bft, *, block_c, hist):
    nrows, embed_dim = gathered.shape
    batch = nrows // hist
    num_out = wft.shape[0]
    per_l = batch // block_c

    def body(x_ref, wft_ref, bft_ref, o_ref):
        xt = x_ref[...].T
        yt = jnp.dot(wft_ref[...], xt, preferred_element_type=jnp.float32)
        o_ref[...] = (yt + bft_ref[...]).reshape(1, num_out, block_c)

    return pl.pallas_call(
        body,
        grid=(hist, per_l),
        in_specs=[
            pl.BlockSpec((block_c, embed_dim), lambda l, j: (l * per_l + j, 0)),
            pl.BlockSpec((num_out, embed_dim), lambda l, j: (0, 0)),
            pl.BlockSpec((num_out, 1), lambda l, j: (0, 0)),
        ],
        out_specs=pl.BlockSpec((1, num_out, block_c), lambda l, j: (l, 0, j)),
        out_shape=jax.ShapeDtypeStruct((hist, num_out, batch), jnp.float32),
    )(gathered, wft, bft)


def kernel(input_notes, emb, W1, b1, W2, b2):
    batch, hist = input_notes.shape
    nrows = batch * hist
    info = plsc.get_sparse_core_info()
    nw = info.num_cores * info.num_subcores
    chunks = nrows // (nw * _CHUNK)

    idx = input_notes.reshape(nrows).astype(jnp.int32)
    rows = jnp.arange(nrows, dtype=jnp.int32)
    sidx = ((rows % hist) * batch + rows // hist).reshape(nw, chunks, _CHUNK)
    sidx = jnp.pad(sidx, ((0, 0), (0, -chunks % 8), (0, 0)))

    gathered = _sc_gather(idx, sidx, emb, chunks=chunks)
    wft, bft = _fuse_weights(W1, W2, b1, b2)
    out_t = _mlp_t(gathered, wft, bft, block_c=2048, hist=hist)
    return jnp.transpose(out_t, (2, 0, 1))


# speedup vs baseline: 4.5774x; 1.0214x over previous
import functools

import jax
import jax.numpy as jnp
from jax import lax
from jax.experimental import pallas as pl
from jax.experimental.pallas import tpu as pltpu
from jax.experimental.pallas import tpu_sc as plsc

_CHUNK = 128


def _sc_gather(idx, sidx, emb, *, chunks):
    nrows = idx.shape[0]
    embed_dim = emb.shape[1]
    pad_chunks = sidx.shape[1]
    mesh = plsc.VectorSubcoreMesh(core_axis_name="c", subcore_axis_name="s")
    num_cores = mesh.num_cores
    nbuf = 4

    @functools.partial(
        pl.kernel,
        out_type=jax.ShapeDtypeStruct((nrows, embed_dim), jnp.float32),
        mesh=mesh,
        scratch_types=[
            pltpu.VMEM((chunks * _CHUNK,), jnp.int32),
            pltpu.VMEM((pad_chunks, _CHUNK), jnp.int32),
            pltpu.VMEM((nbuf, _CHUNK, embed_dim), jnp.float32),
            [pltpu.SemaphoreType.DMA] * nbuf,
            [pltpu.SemaphoreType.DMA] * nbuf,
        ],
    )
    def gather_kernel(emb_hbm, idx_hbm, sidx_hbm, out_hbm, idx_v, sidx_v,
                      rows_v, gsems, ssems):
        wid = lax.axis_index("s") * num_cores + lax.axis_index("c")
        base = wid * (chunks * _CHUNK)
        pltpu.sync_copy(idx_hbm.at[pl.ds(base, chunks * _CHUNK)], idx_v)
        pltpu.sync_copy(sidx_hbm.at[wid], sidx_v)

        def gather(g):
            return pltpu.async_copy(
                emb_hbm.at[idx_v.at[pl.ds(g * _CHUNK, _CHUNK)]],
                rows_v.at[g % nbuf],
                gsems[g % nbuf],
            )

        def scatter(g):
            return pltpu.async_copy(
                rows_v.at[g % nbuf],
                out_hbm.at[sidx_v.at[g]],
                ssems[g % nbuf],
            )

        gath = {g: gather(g) for g in range(min(2, chunks))}
        scat = {}
        for g in range(chunks):
            gath.pop(g).wait()
            scat[g] = scatter(g)
            nx = g + 2
            if nx < chunks:
                if nx - nbuf in scat:
                    scat.pop(nx - nbuf).wait()
                gath[nx] = gather(nx)
        for d in scat.values():
            d.wait()

    return gather_kernel(emb, idx, sidx)


def _fuse_weights(W1, W2, b1, b2):
    def body(w1_ref, w2_ref, b1_ref, b2_ref, wft_ref, bft_ref):
        w2 = w2_ref[...]
        wft_ref[...] = lax.dot_general(
            w2, w1_ref[...],
            dimension_numbers=(((0,), (1,)), ((), ())),
            preferred_element_type=jnp.float32,
            precision=lax.Precision.HIGHEST,
        )
        bft_ref[...] = (
            lax.dot_general(
                w2, b1_ref[...],
                dimension_numbers=(((0,), (0,)), ((), ())),
                preferred_element_type=jnp.float32,
                precision=lax.Precision.HIGHEST,
            )
            + b2_ref[...]
        )

    embed_dim, hidden = W1.shape
    num_out = W2.shape[1]
    return pl.pallas_call(
        body,
        out_shape=(
            jax.ShapeDtypeStruct((num_out, embed_dim), jnp.float32),
            jax.ShapeDtypeStruct((num_out, 1), jnp.float32),
        ),
    )(W1, W2, b1.reshape(hidden, 1), b2.reshape(num_out, 1))


def _mlp_t(gathered, wft, bft, *, block_c, hist):
    nrows, embed_dim = gathered.shape
    batch = nrows // hist
    num_out = wft.shape[0]
    per_l = batch // block_c

    def body(x_ref, wft_ref, bft_ref, o_ref):
        xt = x_ref[...].T
        yt = jnp.dot(wft_ref[...], xt, preferred_element_type=jnp.float32)
        o_ref[...] = (yt + bft_ref[...]).reshape(1, num_out, block_c)

    return pl.pallas_call(
        body,
        grid=(hist, per_l),
        in_specs=[
            pl.BlockSpec((block_c, embed_dim), lambda l, j: (l * per_l + j, 0)),
            pl.BlockSpec((num_out, embed_dim), lambda l, j: (0, 0)),
            pl.BlockSpec((num_out, 1), lambda l, j: (0, 0)),
        ],
        out_specs=pl.BlockSpec((1, num_out, block_c), lambda l, j: (l, 0, j)),
        out_shape=jax.ShapeDtypeStruct((hist, num_out, batch), jnp.float32),
    )(gathered, wft, bft)


def kernel(input_notes, emb, W1, b1, W2, b2):
    batch, hist = input_notes.shape
    nrows = batch * hist
    info = plsc.get_sparse_core_info()
    nw = info.num_cores * info.num_subcores
    chunks = nrows // (nw * _CHUNK)

    idx = input_notes.reshape(nrows).astype(jnp.int32)
    rows = jnp.arange(nrows, dtype=jnp.int32)
    sidx = ((rows % hist) * batch + rows // hist).reshape(nw, chunks, _CHUNK)
    sidx = jnp.pad(sidx, ((0, 0), (0, -chunks % 8), (0, 0)))

    gathered = _sc_gather(idx, sidx, emb, chunks=chunks)
    wft, bft = _fuse_weights(W1, W2, b1, b2)
    out_t = _mlp_t(gathered, wft, bft, block_c=4096, hist=hist)
    return jnp.transpose(out_t, (2, 0, 1))
